# baseline (speedup 1.0000x reference)
import numpy as np

import jax
import jax.numpy as jnp
from jax import lax
from jax.experimental import pallas as pl
from jax.experimental.pallas import tpu as pltpu
from jax.experimental.pallas import tpu_sc as plsc

N = 10000
E = 320000
K = 64
F = 128
FW = F // 2
R = 2

NC = 2
NS = 16
NW = NC * NS
EPT = E // NW
C = 40
CHUNKS = EPT // C
DRAIN_TILES = 10
DRAIN_ROWS = N // DRAIN_TILES

RHO = np.zeros(F, dtype=np.int32)
for _c in range(F // 32):
    for _t in range(16):
        RHO[32 * _c + _t] = 32 * _c + 2 * _t
        RHO[32 * _c + 16 + _t] = 32 * _c + 2 * _t + 1


def _pack_bf16_pair(lo_f32, hi_f32):
    ulo = lax.bitcast_convert_type(lo_f32, jnp.uint32)
    uhi = lax.bitcast_convert_type(hi_f32, jnp.uint32)
    word = ((ulo + 0x8000) >> 16) | ((uhi + 0x8000) & jnp.uint32(0xFFFF0000))
    return lax.bitcast_convert_type(word, jnp.int32)


def _g_body(rbf_ref, we_ref, wo_ref, o_ref):
    rbf_bf = rbf_ref[...].astype(jnp.bfloat16)
    ge = jnp.dot(rbf_bf, we_ref[...], preferred_element_type=jnp.float32)
    go = jnp.dot(rbf_bf, wo_ref[...], preferred_element_type=jnp.float32)
    o_ref[...] = _pack_bf16_pair(ge, go)


def _xj_body(x_ref, w_ref, b_ref, o_ref):
    o_ref[...] = jnp.dot(x_ref[...], w_ref[...],
                         preferred_element_type=jnp.float32) + b_ref[...]


def _tail_body(x_ref, p_ref, wi_ref, bi_ref, w1_ref, b1_ref, w2_ref, b2_ref,
               wd_ref, bd_ref, u_ref, o_ref):
    xb = x_ref[...]
    m = (jnp.dot(xb, wi_ref[...], preferred_element_type=jnp.float32)
         + bi_ref[...] + p_ref[0] + p_ref[1])
    for r in range(R):
        t = jnp.dot(m, w1_ref[r], preferred_element_type=jnp.float32) + b1_ref[r]
        m = m + jnp.dot(t, w2_ref[r], preferred_element_type=jnp.float32) + b2_ref[r]
    o_ref[...] = (u_ref[...] * xb
                  + jnp.dot(m, wd_ref[...], preferred_element_type=jnp.float32)
                  + bd_ref[...])


def _sc_body(g_hbm, xj_hbm, idxi_hbm, idxj_hbm, out_hbm,
             idxj_v, idxi_cb, rows_v, gbuf_v, sbuf_v, acc_sh,
             gsem0, gsem1, csem0, csem1, ssem0, ssem1, isem0, isem1):
    core = lax.axis_index("c")
    sub = lax.axis_index("s")
    tile = core * NS + sub
    sems = ((gsem0, csem0, ssem0, isem0), (gsem1, csem1, ssem1, isem1))
    base0 = tile * EPT

    z16 = jnp.zeros((16,), jnp.float32)

    def zrow(r, carry):
        for c8 in range(F // 16):
            sbuf_v[0, r, pl.ds(c8 * 16, 16)] = z16
        return carry

    lax.fori_loop(0, C, zrow, 0)

    @pl.when(sub < DRAIN_TILES)
    def _zero():
        for part in range(DRAIN_ROWS // C):
            pltpu.async_copy(
                sbuf_v.at[0],
                acc_sh.at[pl.ds(sub * DRAIN_ROWS + part * C, C)], gsem0)
        for part in range(DRAIN_ROWS // C):
            pltpu.make_async_copy(
                sbuf_v.at[0],
                acc_sh.at[pl.ds(sub * DRAIN_ROWS + part * C, C)],
                gsem0).wait()

    pltpu.sync_copy(idxj_hbm.at[pl.ds(pl.multiple_of(base0, 8), EPT)], idxj_v)
    plsc.subcore_barrier()

    def issue(kk, b):
        gs, cs = sems[b][0], sems[b][1]
        pltpu.async_copy(xj_hbm.at[idxj_v.at[pl.ds(kk * C, C)]],
                         rows_v.at[b], gs)
        pltpu.async_copy(g_hbm.at[pl.ds(pl.multiple_of(base0 + kk * C, 8), C)],
                         gbuf_v.at[b], cs)

    def fetch_idxi(kk, b):
        pltpu.async_copy(
            idxi_hbm.at[pl.ds(pl.multiple_of(base0 + kk * C, 8), C)],
            idxi_cb.at[b], sems[b][3])

    issue(0, 0)
    issue(1, 1)
    fetch_idxi(0, 0)
    fetch_idxi(1, 1)

    def pair(kp, carry):
        k0 = kp * 2
        for b in range(2):
            kk = k0 + b
            gs, cs, ss, isem = sems[b]
            pltpu.make_async_copy(
                xj_hbm.at[idxj_v.at[pl.ds(kk * C, C)]], rows_v.at[b],
                gs).wait()
            pltpu.make_async_copy(
                g_hbm.at[pl.ds(pl.multiple_of(base0 + kk * C, 8), C)],
                gbuf_v.at[b], cs).wait()

            @pl.when(kk >= 2)
            def _recycle():
                pltpu.make_async_copy(
                    sbuf_v.at[b], acc_sh.at[idxi_cb.at[b]], ss).wait()
                fetch_idxi(kk, b)

            def mrow(r, inner):
                himask = jnp.int32(-65536)
                for c in range(F // 32):
                    gw = gbuf_v[b, r, pl.ds(c * 16, 16)]
                    ga = lax.bitcast_convert_type(gw << 16, jnp.float32)
                    gb = lax.bitcast_convert_type(gw & himask, jnp.float32)
                    xa = rows_v[b, r, pl.ds(32 * c, 16)]
                    xb2 = rows_v[b, r, pl.ds(32 * c + 16, 16)]
                    sbuf_v[b, r, pl.ds(32 * c, 16)] = ga * xa
                    sbuf_v[b, r, pl.ds(32 * c + 16, 16)] = gb * xb2
                return inner

            lax.fori_loop(0, C, mrow, 0, unroll=2)

            @pl.when(kk + 2 < CHUNKS)
            def _prefetch():
                issue(kk + 2, b)

            pltpu.make_async_copy(
                idxi_hbm.at[pl.ds(pl.multiple_of(base0 + kk * C, 8), C)],
                idxi_cb.at[b], isem).wait()
            pltpu.async_copy(sbuf_v.at[b], acc_sh.at[idxi_cb.at[b]], ss,
                             add=True)
        return carry

    lax.fori_loop(0, CHUNKS // 2, pair, 0)
    for b in range(2):
        pltpu.make_async_copy(
            sbuf_v.at[b], acc_sh.at[idxi_cb.at[b]], sems[b][2]).wait()

    plsc.subcore_barrier()

    @pl.when(sub < DRAIN_TILES)
    def _drain():
        pltpu.sync_copy(
            acc_sh.at[pl.ds(sub * DRAIN_ROWS, DRAIN_ROWS)],
            out_hbm.at[core, pl.ds(sub * DRAIN_ROWS, DRAIN_ROWS)])


def kernel(x, rbf, idx_i, idx_j, Wk2f, Wi, bi, Wj, bj, W1, b1, W2, b2, Wd, bd, u):
    BE = 16000
    BN = 2000
    rho = jnp.asarray(RHO)

    g_packed = pl.pallas_call(
        _g_body,
        grid=(E // BE,),
        in_specs=[
            pl.BlockSpec((BE, K), lambda i: (i, 0)),
            pl.BlockSpec((K, FW), lambda i: (0, 0)),
            pl.BlockSpec((K, FW), lambda i: (0, 0)),
        ],
        out_specs=pl.BlockSpec((BE, FW), lambda i: (i, 0)),
        out_shape=jax.ShapeDtypeStruct((E, FW), jnp.int32),
    )(rbf, Wk2f[:, 0::2].astype(jnp.bfloat16), Wk2f[:, 1::2].astype(jnp.bfloat16))

    xj_perm = pl.pallas_call(
        _xj_body,
        grid=(N // BN,),
        in_specs=[
            pl.BlockSpec((BN, F), lambda i: (i, 0)),
            pl.BlockSpec((F, F), lambda i: (0, 0)),
            pl.BlockSpec((1, F), lambda i: (0, 0)),
        ],
        out_specs=pl.BlockSpec((BN, F), lambda i: (i, 0)),
        out_shape=jax.ShapeDtypeStruct((N, F), jnp.float32),
    )(x, Wj[:, rho], bj[rho].reshape(1, F))

    mesh = plsc.VectorSubcoreMesh(core_axis_name="c", subcore_axis_name="s")
    partials = pl.kernel(
        _sc_body,
        out_type=jax.ShapeDtypeStruct((NC, N, F), jnp.float32),
        mesh=mesh,
        scratch_types=[
            pltpu.VMEM((EPT,), jnp.int32),
            pltpu.VMEM((2, C), jnp.int32),
            pltpu.VMEM((2, C, F), jnp.float32),
            pltpu.VMEM((2, C, FW), jnp.int32),
            pltpu.VMEM((2, C, F), jnp.float32),
            pltpu.VMEM_SHARED((N, F), jnp.float32),
            pltpu.SemaphoreType.DMA,
            pltpu.SemaphoreType.DMA,
            pltpu.SemaphoreType.DMA,
            pltpu.SemaphoreType.DMA,
            pltpu.SemaphoreType.DMA,
            pltpu.SemaphoreType.DMA,
            pltpu.SemaphoreType.DMA,
            pltpu.SemaphoreType.DMA,
        ],
    )(g_packed, xj_perm, idx_i, idx_j)

    out = pl.pallas_call(
        _tail_body,
        grid=(N // BN,),
        in_specs=[
            pl.BlockSpec((BN, F), lambda i: (i, 0)),
            pl.BlockSpec((NC, BN, F), lambda i: (0, i, 0)),
            pl.BlockSpec((F, F), lambda i: (0, 0)),
            pl.BlockSpec((1, F), lambda i: (0, 0)),
            pl.BlockSpec((R, F, F), lambda i: (0, 0, 0)),
            pl.BlockSpec((R, 1, F), lambda i: (0, 0, 0)),
            pl.BlockSpec((R, F, F), lambda i: (0, 0, 0)),
            pl.BlockSpec((R, 1, F), lambda i: (0, 0, 0)),
            pl.BlockSpec((F, F), lambda i: (0, 0)),
            pl.BlockSpec((1, F), lambda i: (0, 0)),
            pl.BlockSpec((1, F), lambda i: (0, 0)),
        ],
        out_specs=pl.BlockSpec((BN, F), lambda i: (i, 0)),
        out_shape=jax.ShapeDtypeStruct((N, F), jnp.float32),
    )(x, partials, Wi[:, rho], bi[rho].reshape(1, F),
      W1[:, rho, :], b1.reshape(R, 1, F),
      W2[:, :, rho], b2[:, rho].reshape(R, 1, F),
      Wd[rho, :], bd.reshape(1, F), u.reshape(1, F))

    return out


# speedup vs baseline: 1.1517x; 1.1517x over previous
import numpy as np

import jax
import jax.numpy as jnp
from jax import lax
from jax.experimental import pallas as pl
from jax.experimental.pallas import tpu as pltpu
from jax.experimental.pallas import tpu_sc as plsc

N = 10000
E = 320000
K = 64
F = 128
FW = F // 2
R = 2

NC = 2
NS = 16
NW = NC * NS
EPT = E // NW
C = 40
CHUNKS = EPT // C
DRAIN_TILES = 10
DRAIN_ROWS = N // DRAIN_TILES

RHO = np.zeros(F, dtype=np.int32)
for _c in range(F // 32):
    for _t in range(16):
        RHO[32 * _c + _t] = 32 * _c + 2 * _t
        RHO[32 * _c + 16 + _t] = 32 * _c + 2 * _t + 1


def _pack_bf16_pair(lo_f32, hi_f32):
    ulo = lax.bitcast_convert_type(lo_f32, jnp.uint32)
    uhi = lax.bitcast_convert_type(hi_f32, jnp.uint32)
    word = ((ulo + 0x8000) >> 16) | ((uhi + 0x8000) & jnp.uint32(0xFFFF0000))
    return lax.bitcast_convert_type(word, jnp.int32)


def _g_body(rbf_ref, we_ref, wo_ref, o_ref):
    rbf_bf = rbf_ref[...].astype(jnp.bfloat16)
    ge = jnp.dot(rbf_bf, we_ref[...], preferred_element_type=jnp.float32)
    go = jnp.dot(rbf_bf, wo_ref[...], preferred_element_type=jnp.float32)
    o_ref[...] = _pack_bf16_pair(ge, go)


def _xj_body(x_ref, w_ref, b_ref, o_ref):
    o_ref[...] = jnp.dot(x_ref[...], w_ref[...],
                         preferred_element_type=jnp.float32) + b_ref[...]


def _tail_body(x_ref, p_ref, wi_ref, bi_ref, w1_ref, b1_ref, w2_ref, b2_ref,
               wd_ref, bd_ref, u_ref, o_ref):
    xb = x_ref[...]
    m = (jnp.dot(xb, wi_ref[...], preferred_element_type=jnp.float32)
         + bi_ref[...] + p_ref[0] + p_ref[1])
    for r in range(R):
        t = jnp.dot(m, w1_ref[r], preferred_element_type=jnp.float32) + b1_ref[r]
        m = m + jnp.dot(t, w2_ref[r], preferred_element_type=jnp.float32) + b2_ref[r]
    o_ref[...] = (u_ref[...] * xb
                  + jnp.dot(m, wd_ref[...], preferred_element_type=jnp.float32)
                  + bd_ref[...])


def _sc_body(g_hbm, xj_hbm, idxi_hbm, idxj_hbm, out_hbm,
             idxj_v, idxi_cb, rows_v, gbuf_v, sbuf_v, acc_sh,
             gsem0, gsem1, csem0, csem1, ssem0, ssem1, isem0, isem1):
    core = lax.axis_index("c")
    sub = lax.axis_index("s")
    tile = core * NS + sub
    sems = ((gsem0, csem0, ssem0, isem0), (gsem1, csem1, ssem1, isem1))
    base0 = tile * EPT

    z16 = jnp.zeros((16,), jnp.float32)

    def zrow(r, carry):
        for c8 in range(F // 16):
            sbuf_v[0, r, pl.ds(c8 * 16, 16)] = z16
        return carry

    lax.fori_loop(0, C, zrow, 0)

    @pl.when(sub < DRAIN_TILES)
    def _zero():
        for part in range(DRAIN_ROWS // C):
            pltpu.async_copy(
                sbuf_v.at[0],
                acc_sh.at[pl.ds(sub * DRAIN_ROWS + part * C, C)], gsem0)
        for part in range(DRAIN_ROWS // C):
            pltpu.make_async_copy(
                sbuf_v.at[0],
                acc_sh.at[pl.ds(sub * DRAIN_ROWS + part * C, C)],
                gsem0).wait()

    pltpu.sync_copy(idxj_hbm.at[pl.ds(pl.multiple_of(base0, 8), EPT)], idxj_v)
    plsc.subcore_barrier()

    def issue(kk, b):
        gs, cs = sems[b][0], sems[b][1]
        pltpu.async_copy(xj_hbm.at[idxj_v.at[pl.ds(kk * C, C)]],
                         rows_v.at[b], gs)
        pltpu.async_copy(g_hbm.at[pl.ds(pl.multiple_of(base0 + kk * C, 8), C)],
                         gbuf_v.at[b], cs)

    def fetch_idxi(kk, b):
        pltpu.async_copy(
            idxi_hbm.at[pl.ds(pl.multiple_of(base0 + kk * C, 8), C)],
            idxi_cb.at[b], sems[b][3])

    issue(0, 0)
    issue(1, 1)
    fetch_idxi(0, 0)
    fetch_idxi(1, 1)

    def pair(kp, carry):
        k0 = kp * 2
        for b in range(2):
            kk = k0 + b
            gs, cs, ss, isem = sems[b]
            pltpu.make_async_copy(
                xj_hbm.at[idxj_v.at[pl.ds(kk * C, C)]], rows_v.at[b],
                gs).wait()
            pltpu.make_async_copy(
                g_hbm.at[pl.ds(pl.multiple_of(base0 + kk * C, 8), C)],
                gbuf_v.at[b], cs).wait()

            @pl.when(kk >= 2)
            def _recycle():
                pltpu.make_async_copy(
                    sbuf_v.at[b], acc_sh.at[idxi_cb.at[b]], ss).wait()
                fetch_idxi(kk, b)

            @plsc.parallel_loop(0, C, step=1)
            def mrow(r):
                himask = jnp.int32(-65536)
                for c in range(F // 32):
                    gw = gbuf_v[b, r, pl.ds(c * 16, 16)]
                    ga = lax.bitcast_convert_type(gw << 16, jnp.float32)
                    gb = lax.bitcast_convert_type(gw & himask, jnp.float32)
                    xa = rows_v[b, r, pl.ds(32 * c, 16)]
                    xb2 = rows_v[b, r, pl.ds(32 * c + 16, 16)]
                    sbuf_v[b, r, pl.ds(32 * c, 16)] = ga * xa
                    sbuf_v[b, r, pl.ds(32 * c + 16, 16)] = gb * xb2

            @pl.when(kk + 2 < CHUNKS)
            def _prefetch():
                issue(kk + 2, b)

            pltpu.make_async_copy(
                idxi_hbm.at[pl.ds(pl.multiple_of(base0 + kk * C, 8), C)],
                idxi_cb.at[b], isem).wait()
            pltpu.async_copy(sbuf_v.at[b], acc_sh.at[idxi_cb.at[b]], ss,
                             add=True)
        return carry

    lax.fori_loop(0, CHUNKS // 2, pair, 0)
    for b in range(2):
        pltpu.make_async_copy(
            sbuf_v.at[b], acc_sh.at[idxi_cb.at[b]], sems[b][2]).wait()

    plsc.subcore_barrier()

    @pl.when(sub < DRAIN_TILES)
    def _drain():
        pltpu.sync_copy(
            acc_sh.at[pl.ds(sub * DRAIN_ROWS, DRAIN_ROWS)],
            out_hbm.at[core, pl.ds(sub * DRAIN_ROWS, DRAIN_ROWS)])


def kernel(x, rbf, idx_i, idx_j, Wk2f, Wi, bi, Wj, bj, W1, b1, W2, b2, Wd, bd, u):
    BE = 16000
    BN = 2000
    rho = jnp.asarray(RHO)

    g_packed = pl.pallas_call(
        _g_body,
        grid=(E // BE,),
        in_specs=[
            pl.BlockSpec((BE, K), lambda i: (i, 0)),
            pl.BlockSpec((K, FW), lambda i: (0, 0)),
            pl.BlockSpec((K, FW), lambda i: (0, 0)),
        ],
        out_specs=pl.BlockSpec((BE, FW), lambda i: (i, 0)),
        out_shape=jax.ShapeDtypeStruct((E, FW), jnp.int32),
    )(rbf, Wk2f[:, 0::2].astype(jnp.bfloat16), Wk2f[:, 1::2].astype(jnp.bfloat16))

    xj_perm = pl.pallas_call(
        _xj_body,
        grid=(N // BN,),
        in_specs=[
            pl.BlockSpec((BN, F), lambda i: (i, 0)),
            pl.BlockSpec((F, F), lambda i: (0, 0)),
            pl.BlockSpec((1, F), lambda i: (0, 0)),
        ],
        out_specs=pl.BlockSpec((BN, F), lambda i: (i, 0)),
        out_shape=jax.ShapeDtypeStruct((N, F), jnp.float32),
    )(x, Wj[:, rho], bj[rho].reshape(1, F))

    mesh = plsc.VectorSubcoreMesh(core_axis_name="c", subcore_axis_name="s")
    partials = pl.kernel(
        _sc_body,
        out_type=jax.ShapeDtypeStruct((NC, N, F), jnp.float32),
        mesh=mesh,
        scratch_types=[
            pltpu.VMEM((EPT,), jnp.int32),
            pltpu.VMEM((2, C), jnp.int32),
            pltpu.VMEM((2, C, F), jnp.float32),
            pltpu.VMEM((2, C, FW), jnp.int32),
            pltpu.VMEM((2, C, F), jnp.float32),
            pltpu.VMEM_SHARED((N, F), jnp.float32),
            pltpu.SemaphoreType.DMA,
            pltpu.SemaphoreType.DMA,
            pltpu.SemaphoreType.DMA,
            pltpu.SemaphoreType.DMA,
            pltpu.SemaphoreType.DMA,
            pltpu.SemaphoreType.DMA,
            pltpu.SemaphoreType.DMA,
            pltpu.SemaphoreType.DMA,
        ],
    )(g_packed, xj_perm, idx_i, idx_j)

    out = pl.pallas_call(
        _tail_body,
        grid=(N // BN,),
        in_specs=[
            pl.BlockSpec((BN, F), lambda i: (i, 0)),
            pl.BlockSpec((NC, BN, F), lambda i: (0, i, 0)),
            pl.BlockSpec((F, F), lambda i: (0, 0)),
            pl.BlockSpec((1, F), lambda i: (0, 0)),
            pl.BlockSpec((R, F, F), lambda i: (0, 0, 0)),
            pl.BlockSpec((R, 1, F), lambda i: (0, 0, 0)),
            pl.BlockSpec((R, F, F), lambda i: (0, 0, 0)),
            pl.BlockSpec((R, 1, F), lambda i: (0, 0, 0)),
            pl.BlockSpec((F, F), lambda i: (0, 0)),
            pl.BlockSpec((1, F), lambda i: (0, 0)),
            pl.BlockSpec((1, F), lambda i: (0, 0)),
        ],
        out_specs=pl.BlockSpec((BN, F), lambda i: (i, 0)),
        out_shape=jax.ShapeDtypeStruct((N, F), jnp.float32),
    )(x, partials, Wi[:, rho], bi[rho].reshape(1, F),
      W1[:, rho, :], b1.reshape(R, 1, F),
      W2[:, :, rho], b2[:, rho].reshape(R, 1, F),
      Wd[rho, :], bd.reshape(1, F), u.reshape(1, F))

    return out


# speedup vs baseline: 1.1534x; 1.0014x over previous
import numpy as np

import jax
import jax.numpy as jnp
from jax import lax
from jax.experimental import pallas as pl
from jax.experimental.pallas import tpu as pltpu
from jax.experimental.pallas import tpu_sc as plsc

N = 10000
E = 320000
K = 64
F = 128
FW = F // 2
R = 2

NC = 2
NS = 16
NW = NC * NS
EPT = E // NW
C = 40
CHUNKS = EPT // C
DRAIN_TILES = 10
DRAIN_ROWS = N // DRAIN_TILES

RHO = np.zeros(F, dtype=np.int32)
for _c in range(F // 32):
    for _t in range(16):
        RHO[32 * _c + _t] = 32 * _c + 2 * _t
        RHO[32 * _c + 16 + _t] = 32 * _c + 2 * _t + 1


def _pack_bf16_pair(lo_f32, hi_f32):
    ulo = lax.bitcast_convert_type(lo_f32, jnp.uint32)
    uhi = lax.bitcast_convert_type(hi_f32, jnp.uint32)
    word = ((ulo + 0x8000) >> 16) | ((uhi + 0x8000) & jnp.uint32(0xFFFF0000))
    return lax.bitcast_convert_type(word, jnp.int32)


def _g_body(rbf_ref, we_ref, wo_ref, o_ref):
    rbf_bf = rbf_ref[...].astype(jnp.bfloat16)
    ge = jnp.dot(rbf_bf, we_ref[...], preferred_element_type=jnp.float32)
    go = jnp.dot(rbf_bf, wo_ref[...], preferred_element_type=jnp.float32)
    o_ref[...] = _pack_bf16_pair(ge, go)


def _xj_body(x_ref, w_ref, b_ref, o_ref):
    o_ref[...] = jnp.dot(x_ref[...], w_ref[...],
                         preferred_element_type=jnp.float32) + b_ref[...]


def _tail_body(x_ref, p_ref, wi_ref, bi_ref, w1_ref, b1_ref, w2_ref, b2_ref,
               wd_ref, bd_ref, u_ref, o_ref):
    xb = x_ref[...]
    m = (jnp.dot(xb, wi_ref[...], preferred_element_type=jnp.float32)
         + bi_ref[...] + p_ref[0] + p_ref[1])
    for r in range(R):
        t = jnp.dot(m, w1_ref[r], preferred_element_type=jnp.float32) + b1_ref[r]
        m = m + jnp.dot(t, w2_ref[r], preferred_element_type=jnp.float32) + b2_ref[r]
    o_ref[...] = (u_ref[...] * xb
                  + jnp.dot(m, wd_ref[...], preferred_element_type=jnp.float32)
                  + bd_ref[...])


def _sc_body(g_hbm, xj_hbm, idxi_hbm, idxj_hbm, out_hbm,
             idxj_v, idxi_cb, rows_v, gbuf_v, sbuf_v, acc_sh,
             gsem0, gsem1, csem0, csem1, ssem0, ssem1, isem0, isem1):
    core = lax.axis_index("c")
    sub = lax.axis_index("s")
    tile = core * NS + sub
    sems = ((gsem0, csem0, ssem0, isem0), (gsem1, csem1, ssem1, isem1))
    base0 = tile * EPT

    z16 = jnp.zeros((16,), jnp.float32)

    def zrow(r, carry):
        for c8 in range(F // 16):
            sbuf_v[0, r, pl.ds(c8 * 16, 16)] = z16
        return carry

    lax.fori_loop(0, C, zrow, 0)

    @pl.when(sub < DRAIN_TILES)
    def _zero():
        for part in range(DRAIN_ROWS // C):
            pltpu.async_copy(
                sbuf_v.at[0],
                acc_sh.at[pl.ds(sub * DRAIN_ROWS + part * C, C)], gsem0)
        for part in range(DRAIN_ROWS // C):
            pltpu.make_async_copy(
                sbuf_v.at[0],
                acc_sh.at[pl.ds(sub * DRAIN_ROWS + part * C, C)],
                gsem0).wait()

    pltpu.sync_copy(idxj_hbm.at[pl.ds(pl.multiple_of(base0, 8), EPT)], idxj_v)
    plsc.subcore_barrier()

    def issue(kk, b):
        gs, cs = sems[b][0], sems[b][1]
        pltpu.async_copy(xj_hbm.at[idxj_v.at[pl.ds(kk * C, C)]],
                         rows_v.at[b], gs)
        pltpu.async_copy(g_hbm.at[pl.ds(pl.multiple_of(base0 + kk * C, 8), C)],
                         gbuf_v.at[b], cs)

    def fetch_idxi(kk, b):
        pltpu.async_copy(
            idxi_hbm.at[pl.ds(pl.multiple_of(base0 + kk * C, 8), C)],
            idxi_cb.at[b], sems[b][3])

    issue(0, 0)
    issue(1, 1)
    fetch_idxi(0, 0)
    fetch_idxi(1, 1)

    def pair(kp, carry):
        k0 = kp * 2
        for b in range(2):
            kk = k0 + b
            gs, cs, ss, isem = sems[b]
            pltpu.make_async_copy(
                xj_hbm.at[idxj_v.at[pl.ds(kk * C, C)]], rows_v.at[b],
                gs).wait()
            pltpu.make_async_copy(
                g_hbm.at[pl.ds(pl.multiple_of(base0 + kk * C, 8), C)],
                gbuf_v.at[b], cs).wait()

            @pl.when(kk >= 2)
            def _recycle():
                pltpu.make_async_copy(
                    sbuf_v.at[b], acc_sh.at[idxi_cb.at[b]], ss).wait()
                fetch_idxi(kk, b)

            @plsc.parallel_loop(0, C, step=1)
            def mrow(r):
                himask = jnp.int32(-65536)
                for c in range(F // 32):
                    gw = gbuf_v[b, r, pl.ds(c * 16, 16)]
                    ga = lax.bitcast_convert_type(gw << 16, jnp.float32)
                    gb = lax.bitcast_convert_type(gw & himask, jnp.float32)
                    xa = rows_v[b, r, pl.ds(32 * c, 16)]
                    xb2 = rows_v[b, r, pl.ds(32 * c + 16, 16)]
                    sbuf_v[b, r, pl.ds(32 * c, 16)] = ga * xa
                    sbuf_v[b, r, pl.ds(32 * c + 16, 16)] = gb * xb2

            @pl.when(kk + 2 < CHUNKS)
            def _prefetch():
                issue(kk + 2, b)

            pltpu.make_async_copy(
                idxi_hbm.at[pl.ds(pl.multiple_of(base0 + kk * C, 8), C)],
                idxi_cb.at[b], isem).wait()
            pltpu.async_copy(sbuf_v.at[b], acc_sh.at[idxi_cb.at[b]], ss,
                             add=True)
        return carry

    lax.fori_loop(0, CHUNKS // 2, pair, 0)
    for b in range(2):
        pltpu.make_async_copy(
            sbuf_v.at[b], acc_sh.at[idxi_cb.at[b]], sems[b][2]).wait()

    plsc.subcore_barrier()

    @pl.when(sub < DRAIN_TILES)
    def _drain():
        pltpu.sync_copy(
            acc_sh.at[pl.ds(sub * DRAIN_ROWS, DRAIN_ROWS)],
            out_hbm.at[core, pl.ds(sub * DRAIN_ROWS, DRAIN_ROWS)])


def kernel(x, rbf, idx_i, idx_j, Wk2f, Wi, bi, Wj, bj, W1, b1, W2, b2, Wd, bd, u):
    BE = 16000
    BN = 5000
    rho = jnp.asarray(RHO)

    g_packed = pl.pallas_call(
        _g_body,
        grid=(E // BE,),
        in_specs=[
            pl.BlockSpec((BE, K), lambda i: (i, 0)),
            pl.BlockSpec((K, FW), lambda i: (0, 0)),
            pl.BlockSpec((K, FW), lambda i: (0, 0)),
        ],
        out_specs=pl.BlockSpec((BE, FW), lambda i: (i, 0)),
        out_shape=jax.ShapeDtypeStruct((E, FW), jnp.int32),
    )(rbf, Wk2f[:, 0::2].astype(jnp.bfloat16), Wk2f[:, 1::2].astype(jnp.bfloat16))

    xj_perm = pl.pallas_call(
        _xj_body,
        grid=(N // BN,),
        in_specs=[
            pl.BlockSpec((BN, F), lambda i: (i, 0)),
            pl.BlockSpec((F, F), lambda i: (0, 0)),
            pl.BlockSpec((1, F), lambda i: (0, 0)),
        ],
        out_specs=pl.BlockSpec((BN, F), lambda i: (i, 0)),
        out_shape=jax.ShapeDtypeStruct((N, F), jnp.float32),
    )(x, Wj[:, rho], bj[rho].reshape(1, F))

    mesh = plsc.VectorSubcoreMesh(core_axis_name="c", subcore_axis_name="s")
    partials = pl.kernel(
        _sc_body,
        out_type=jax.ShapeDtypeStruct((NC, N, F), jnp.float32),
        mesh=mesh,
        scratch_types=[
            pltpu.VMEM((EPT,), jnp.int32),
            pltpu.VMEM((2, C), jnp.int32),
            pltpu.VMEM((2, C, F), jnp.float32),
            pltpu.VMEM((2, C, FW), jnp.int32),
            pltpu.VMEM((2, C, F), jnp.float32),
            pltpu.VMEM_SHARED((N, F), jnp.float32),
            pltpu.SemaphoreType.DMA,
            pltpu.SemaphoreType.DMA,
            pltpu.SemaphoreType.DMA,
            pltpu.SemaphoreType.DMA,
            pltpu.SemaphoreType.DMA,
            pltpu.SemaphoreType.DMA,
            pltpu.SemaphoreType.DMA,
            pltpu.SemaphoreType.DMA,
        ],
    )(g_packed, xj_perm, idx_i, idx_j)

    out = pl.pallas_call(
        _tail_body,
        grid=(N // BN,),
        in_specs=[
            pl.BlockSpec((BN, F), lambda i: (i, 0)),
            pl.BlockSpec((NC, BN, F), lambda i: (0, i, 0)),
            pl.BlockSpec((F, F), lambda i: (0, 0)),
            pl.BlockSpec((1, F), lambda i: (0, 0)),
            pl.BlockSpec((R, F, F), lambda i: (0, 0, 0)),
            pl.BlockSpec((R, 1, F), lambda i: (0, 0, 0)),
            pl.BlockSpec((R, F, F), lambda i: (0, 0, 0)),
            pl.BlockSpec((R, 1, F), lambda i: (0, 0, 0)),
            pl.BlockSpec((F, F), lambda i: (0, 0)),
            pl.BlockSpec((1, F), lambda i: (0, 0)),
            pl.BlockSpec((1, F), lambda i: (0, 0)),
        ],
        out_specs=pl.BlockSpec((BN, F), lambda i: (i, 0)),
        out_shape=jax.ShapeDtypeStruct((N, F), jnp.float32),
    )(x, partials, Wi[:, rho], bi[rho].reshape(1, F),
      W1[:, rho, :], b1.reshape(R, 1, F),
      W2[:, :, rho], b2[:, rho].reshape(R, 1, F),
      Wd[rho, :], bd.reshape(1, F), u.reshape(1, F))

    return out
